# baseline (speedup 1.0000x reference)
import dataclasses
import functools

import jax
import jax.numpy as jnp
from jax import lax
from jax.experimental import pallas as pl
from jax.experimental.pallas import tpu as pltpu
from jax.experimental.pallas import tpu_sc as plsc

B = 16384
II = 256
IC = 128
DI = 128
NC = 2
NS = 16
NW = NC * NS
BPW = B // NW
CH = 64
NCH = BPW // CH
L = 16


def _fold_body(wt_ref, uvw_ref, wi_ref, out_ref):
    w_full = jnp.dot(wt_ref[...], uvw_ref[...],
                     preferred_element_type=jnp.float32)
    w_if = w_full[384:512, :]
    v_item = jnp.dot(wi_ref[...], w_if,
                     preferred_element_type=jnp.float32)
    out_ref[...] = jnp.concatenate(
        [v_item, w_full[256:384, :], w_full[512:640, :]], axis=0)


_fold = pl.pallas_call(
    _fold_body,
    out_shape=jax.ShapeDtypeStruct((II + DI + IC, 1), jnp.float32),
)


def _score_body(ids_hbm, if_hbm, cf_hbm, emb_hbm, wv_hbm,
                vals_out, ids_out,
                wv_v, idx_v, if_v, cf_v, emb_v, resv_v, resi_v):
    c = lax.axis_index("c")
    s = lax.axis_index("s")
    wid = s * NC + c
    base = wid * BPW

    pltpu.sync_copy(wv_hbm, wv_v)
    pltpu.sync_copy(ids_hbm.at[pl.ds(wid * NCH, NCH)], idx_v)

    wregs = [wv_v[pl.ds(L * j, L)] for j in range(32)]

    best_val = jnp.float32(-3.0e38)
    best_rel = jnp.int32(0)

    for ch in range(NCH):
        row0 = base + ch * CH
        pltpu.sync_copy(if_hbm.at[pl.ds(row0, CH)], if_v)
        pltpu.sync_copy(cf_hbm.at[pl.ds(row0, CH)], cf_v)
        pltpu.sync_copy(emb_hbm.at[idx_v.at[ch]], emb_v)

        def item_step(i, carry, _ch=ch):
            bv, br = carry
            prods = [if_v[i, pl.ds(L * j, L)] * wregs[j] for j in range(16)]
            prods += [emb_v[i, pl.ds(L * j, L)] * wregs[16 + j]
                      for j in range(8)]
            prods += [cf_v[i, pl.ds(L * j, L)] * wregs[24 + j]
                      for j in range(8)]
            while len(prods) > 1:
                prods = [prods[k] + prods[k + 1]
                         for k in range(0, len(prods) - 1, 2)] + (
                             [prods[-1]] if len(prods) % 2 else [])
            sc = jnp.sum(prods[0])
            take = sc > bv
            bv = jnp.where(take, sc, bv)
            br = jnp.where(take, jnp.int32(_ch * CH) + i, br)
            return bv, br

        best_val, best_rel = plsc.parallel_loop(
            0, CH, 1, unroll=2, carry=(best_val, best_rel))(item_step)

    rel_ch = best_rel // CH
    rel_i = best_rel - rel_ch * CH
    win_id = plsc.load_gather(
        idx_v, [jnp.full((L,), rel_ch, jnp.int32),
                jnp.full((L,), rel_i, jnp.int32)])
    resi_v[...] = win_id
    resv_v[...] = jnp.full((L,), best_val, jnp.float32)
    pltpu.sync_copy(resv_v, vals_out.at[wid])
    pltpu.sync_copy(resi_v, ids_out.at[wid])


_sc_params = pltpu.CompilerParams()
if "needs_layout_passes" in pltpu.CompilerParams.__dataclass_fields__:
    _sc_params = dataclasses.replace(_sc_params, needs_layout_passes=False)

_score = functools.partial(
    pl.kernel,
    out_type=(jax.ShapeDtypeStruct((NW, L), jnp.float32),
              jax.ShapeDtypeStruct((NW, L), jnp.int32)),
    mesh=plsc.VectorSubcoreMesh(core_axis_name="c", subcore_axis_name="s"),
    compiler_params=_sc_params,
    scratch_types=[
        pltpu.VMEM((II + DI + IC,), jnp.float32),
        pltpu.VMEM((NCH, CH), jnp.int32),
        pltpu.VMEM((CH, II), jnp.float32),
        pltpu.VMEM((CH, IC), jnp.float32),
        pltpu.VMEM((CH, DI), jnp.float32),
        pltpu.VMEM((L,), jnp.float32),
        pltpu.VMEM((L,), jnp.int32),
    ],
)(_score_body)


def _merge_body(vals_ref, ids_ref, out_ref):
    vals = vals_ref[...]
    ids = ids_ref[...]
    m = jnp.max(vals)
    rows = lax.broadcasted_iota(jnp.int32, (NW, L), 0)
    sel = jnp.where(vals >= m, rows, jnp.int32(2 ** 30))
    r = jnp.min(sel)
    win = jnp.max(jnp.where((vals >= m) & (rows == r), ids,
                            jnp.int32(-2 ** 31)))
    out_ref[...] = jnp.full((1, 1), win, jnp.int32)


_merge = pl.pallas_call(
    _merge_body,
    out_shape=jax.ShapeDtypeStruct((1, 1), jnp.int32),
)


def kernel(user_id, user_features, item_ids, item_features, cross_features,
           user_emb_table, item_emb_table, pos_emb_table,
           W_user, b_user, W_item, b_item, W_task, b_task,
           user_value_weights):
    ids32 = item_ids.astype(jnp.int32)
    folded = _fold(W_task, user_value_weights.reshape(-1, 1), W_item)
    vals, cand_ids = _score(
        ids32.reshape(B // CH, CH), item_features, cross_features,
        item_emb_table, folded.reshape(-1))
    out = _merge(vals, cand_ids)
    return out.reshape(()).astype(item_ids.dtype)


# speedup vs baseline: 1.1155x; 1.1155x over previous
import dataclasses
import functools

import jax
import jax.numpy as jnp
from jax import lax
from jax.experimental import pallas as pl
from jax.experimental.pallas import tpu as pltpu
from jax.experimental.pallas import tpu_sc as plsc

B = 16384
II = 256
IC = 128
DI = 128
NC = 2
NS = 16
NW = NC * NS
BPW = B // NW
CH = 64
NCH = BPW // CH
L = 16


def _fold_body(wt_ref, uvw_ref, wi_ref, out_ref):
    w_full = jnp.dot(wt_ref[...], uvw_ref[...],
                     preferred_element_type=jnp.float32)
    w_if = w_full[384:512, :]
    v_item = jnp.dot(wi_ref[...], w_if,
                     preferred_element_type=jnp.float32)
    out_ref[...] = jnp.concatenate(
        [v_item, w_full[256:384, :], w_full[512:640, :]], axis=0)


_fold = pl.pallas_call(
    _fold_body,
    out_shape=jax.ShapeDtypeStruct((II + DI + IC, 1), jnp.float32),
)


def _score_body(ids_hbm, if_hbm, cf_hbm, emb_hbm, wv_hbm,
                vals_out, ids_out,
                wv_v, idx_v, if_v, cf_v, emb_v, resv_v, resi_v):
    c = lax.axis_index("c")
    s = lax.axis_index("s")
    wid = s * NC + c
    base = wid * BPW

    pltpu.sync_copy(wv_hbm, wv_v)
    pltpu.sync_copy(ids_hbm.at[pl.ds(wid * NCH, NCH)], idx_v)

    wregs = [wv_v[pl.ds(L * j, L)] for j in range(32)]

    best_val = jnp.float32(-3.0e38)
    best_rel = jnp.int32(0)

    for ch in range(NCH):
        row0 = base + ch * CH
        pltpu.sync_copy(if_hbm.at[pl.ds(row0, CH)], if_v)
        pltpu.sync_copy(cf_hbm.at[pl.ds(row0, CH)], cf_v)
        pltpu.sync_copy(emb_hbm.at[idx_v.at[ch]], emb_v)

        def item_step(i, carry, _ch=ch):
            bv, br = carry
            prods = [if_v[i, pl.ds(L * j, L)] * wregs[j] for j in range(16)]
            prods += [emb_v[i, pl.ds(L * j, L)] * wregs[16 + j]
                      for j in range(8)]
            prods += [cf_v[i, pl.ds(L * j, L)] * wregs[24 + j]
                      for j in range(8)]
            while len(prods) > 1:
                prods = [prods[k] + prods[k + 1]
                         for k in range(0, len(prods) - 1, 2)] + (
                             [prods[-1]] if len(prods) % 2 else [])
            sc = jnp.sum(prods[0])
            take = sc > bv
            bv = jnp.where(take, sc, bv)
            br = jnp.where(take, jnp.int32(_ch * CH) + i, br)
            return bv, br

        best_val, best_rel = plsc.parallel_loop(
            0, CH, 1, unroll=1, carry=(best_val, best_rel))(item_step)

    rel_ch = best_rel // CH
    rel_i = best_rel - rel_ch * CH
    win_id = plsc.load_gather(
        idx_v, [jnp.full((L,), rel_ch, jnp.int32),
                jnp.full((L,), rel_i, jnp.int32)])
    resi_v[...] = win_id
    resv_v[...] = jnp.full((L,), best_val, jnp.float32)
    pltpu.sync_copy(resv_v, vals_out.at[wid])
    pltpu.sync_copy(resi_v, ids_out.at[wid])


_sc_params = pltpu.CompilerParams()
if "needs_layout_passes" in pltpu.CompilerParams.__dataclass_fields__:
    _sc_params = dataclasses.replace(_sc_params, needs_layout_passes=False)

_score = functools.partial(
    pl.kernel,
    out_type=(jax.ShapeDtypeStruct((NW, L), jnp.float32),
              jax.ShapeDtypeStruct((NW, L), jnp.int32)),
    mesh=plsc.VectorSubcoreMesh(core_axis_name="c", subcore_axis_name="s"),
    compiler_params=_sc_params,
    scratch_types=[
        pltpu.VMEM((II + DI + IC,), jnp.float32),
        pltpu.VMEM((NCH, CH), jnp.int32),
        pltpu.VMEM((CH, II), jnp.float32),
        pltpu.VMEM((CH, IC), jnp.float32),
        pltpu.VMEM((CH, DI), jnp.float32),
        pltpu.VMEM((L,), jnp.float32),
        pltpu.VMEM((L,), jnp.int32),
    ],
)(_score_body)


def _merge_body(vals_ref, ids_ref, out_ref):
    vals = vals_ref[...]
    ids = ids_ref[...]
    m = jnp.max(vals)
    rows = lax.broadcasted_iota(jnp.int32, (NW, L), 0)
    sel = jnp.where(vals >= m, rows, jnp.int32(2 ** 30))
    r = jnp.min(sel)
    win = jnp.max(jnp.where((vals >= m) & (rows == r), ids,
                            jnp.int32(-2 ** 31)))
    out_ref[...] = jnp.full((1, 1), win, jnp.int32)


_merge = pl.pallas_call(
    _merge_body,
    out_shape=jax.ShapeDtypeStruct((1, 1), jnp.int32),
)


def kernel(user_id, user_features, item_ids, item_features, cross_features,
           user_emb_table, item_emb_table, pos_emb_table,
           W_user, b_user, W_item, b_item, W_task, b_task,
           user_value_weights):
    ids32 = item_ids.astype(jnp.int32)
    folded = _fold(W_task, user_value_weights.reshape(-1, 1), W_item)
    vals, cand_ids = _score(
        ids32.reshape(B // CH, CH), item_features, cross_features,
        item_emb_table, folded.reshape(-1))
    out = _merge(vals, cand_ids)
    return out.reshape(()).astype(item_ids.dtype)


# speedup vs baseline: 1.6212x; 1.4534x over previous
import dataclasses
import functools

import jax
import jax.numpy as jnp
from jax import lax
from jax.experimental import pallas as pl
from jax.experimental.pallas import tpu as pltpu
from jax.experimental.pallas import tpu_sc as plsc

B = 16384
II = 256
IC = 128
DI = 128
NC = 2
NS = 16
NW = NC * NS
BPW = B // NW
CH = 128
NCH = BPW // CH
L = 16
BLK = 2048
CONCAT = 648


def _dense_body(wt_ref, uvw_ref, wi_ref, if_ref, cf_ref, out_ref):
    w_full = jnp.dot(wt_ref[...], uvw_ref[...],
                     preferred_element_type=jnp.float32)
    v_item = jnp.dot(wi_ref[...], w_full[384:512, :],
                     preferred_element_type=jnp.float32)
    s = jnp.dot(if_ref[...], v_item, preferred_element_type=jnp.float32)
    s = s + jnp.dot(cf_ref[...], w_full[512:640, :],
                    preferred_element_type=jnp.float32)
    out_ref[...] = s


_dense = pl.pallas_call(
    _dense_body,
    grid=(B // BLK,),
    in_specs=[
        pl.BlockSpec((CONCAT, 5), lambda i: (0, 0)),
        pl.BlockSpec((5, 1), lambda i: (0, 0)),
        pl.BlockSpec((II, DI), lambda i: (0, 0)),
        pl.BlockSpec((BLK, II), lambda i: (i, 0)),
        pl.BlockSpec((BLK, IC), lambda i: (i, 0)),
    ],
    out_specs=pl.BlockSpec((BLK, 1), lambda i: (i, 0)),
    out_shape=jax.ShapeDtypeStruct((B, 1), jnp.float32),
)


def _emb_body(ids_hbm, emb_hbm, wt_hbm, uvw_hbm,
              out_hbm,
              wt_v, uvw_v, idx_v, emb_v, sco_v):
    c = lax.axis_index("c")
    s = lax.axis_index("s")
    wid = s * NC + c
    base = wid * BPW

    pltpu.sync_copy(wt_hbm, wt_v)
    pltpu.sync_copy(uvw_hbm, uvv := uvw_v)
    pltpu.sync_copy(ids_hbm.at[pl.ds(wid * NCH, NCH)], idx_v)

    wregs = []
    for j in range(8):
        acc = None
        for t in range(5):
            u_t = plsc.load_gather(uvv, [jnp.full((L,), t, jnp.int32)])
            term = wt_v[t, pl.ds(256 + L * j, L)] * u_t
            acc = term if acc is None else acc + term
        wregs.append(acc)

    lane0 = lax.iota(jnp.int32, L) == 0

    for ch in range(NCH):
        pltpu.sync_copy(emb_hbm.at[idx_v.at[ch]], emb_v)

        @plsc.parallel_loop(0, CH, 1, unroll=2)
        def _(i, _ch=ch):
            prods = [emb_v[i, pl.ds(L * j, L)] * wregs[j] for j in range(8)]
            while len(prods) > 1:
                prods = [prods[k] + prods[k + 1]
                         for k in range(0, len(prods), 2)]
            sc = jnp.sum(prods[0])
            plsc.store_scatter(sco_v, [jnp.full((L,), _ch * CH + i,
                                                 jnp.int32)],
                               jnp.full((L,), sc, jnp.float32), mask=lane0)

    pltpu.sync_copy(sco_v, out_hbm.at[pl.ds(base, BPW)])


_sc_params = pltpu.CompilerParams()
if "needs_layout_passes" in pltpu.CompilerParams.__dataclass_fields__:
    _sc_params = dataclasses.replace(_sc_params, needs_layout_passes=False)

_emb_score = functools.partial(
    pl.kernel,
    out_type=jax.ShapeDtypeStruct((B,), jnp.float32),
    mesh=plsc.VectorSubcoreMesh(core_axis_name="c", subcore_axis_name="s"),
    compiler_params=_sc_params,
    scratch_types=[
        pltpu.VMEM((5, CONCAT), jnp.float32),
        pltpu.VMEM((L,), jnp.float32),
        pltpu.VMEM((NCH, CH), jnp.int32),
        pltpu.VMEM((CH, DI), jnp.float32),
        pltpu.VMEM((BPW,), jnp.float32),
    ],
)(_emb_body)


def _final_body(sd_ref, se_ref, ids_ref, out_ref):
    sc = sd_ref[...] + se_ref[...]
    m = jnp.max(sc)
    lin = (lax.broadcasted_iota(jnp.int32, sc.shape, 0) * sc.shape[1]
           + lax.broadcasted_iota(jnp.int32, sc.shape, 1))
    sel = jnp.where(sc >= m, lin, jnp.int32(2 ** 30))
    r = jnp.min(sel)
    win = jnp.max(jnp.where(lin == r, ids_ref[...], jnp.int32(-2 ** 31)))
    out_ref[...] = jnp.full((1, 1), win, jnp.int32)


_final = pl.pallas_call(
    _final_body,
    out_shape=jax.ShapeDtypeStruct((1, 1), jnp.int32),
)


def kernel(user_id, user_features, item_ids, item_features, cross_features,
           user_emb_table, item_emb_table, pos_emb_table,
           W_user, b_user, W_item, b_item, W_task, b_task,
           user_value_weights):
    ids32 = item_ids.astype(jnp.int32)
    uvw16 = jnp.pad(user_value_weights, (0, 11))
    s_dense = _dense(W_task, user_value_weights.reshape(-1, 1), W_item,
                     item_features, cross_features)
    s_emb = _emb_score(ids32.reshape(B // CH, CH), item_emb_table,
                       W_task.T, uvw16)
    out = _final(s_dense.reshape(128, 128), s_emb.reshape(128, 128),
                 ids32.reshape(128, 128))
    return out.reshape(()).astype(item_ids.dtype)
